# baseline (speedup 1.0000x reference)
import functools

import jax
import jax.numpy as jnp
from jax import lax
from jax.experimental import pallas as pl
from jax.experimental.pallas import tpu as pltpu
from jax.experimental.pallas import tpu_sc as plsc

N_NODES = 10000
D = 128
N_EDGES = 320000

NC = 2
NS = 16
L = 16
NW = NC * NS

CHUNK = 128
NBUF = 3
NIB = 5
COL, ROW, EV = 0, 1, 2
CPW0 = 70
CPW1 = 88
CPW_MAX = max(CPW0, CPW1)
_SEGS = [CPW0 * CHUNK if w % NC == 0 else CPW1 * CHUNK for w in range(NW)]
E_PAD = sum(_SEGS)
assert E_PAD >= N_EDGES

TROWS = (N_NODES // NS) // 8 * 8
TAIL = N_NODES - NS * TROWS


def _sc_body(x_hbm, edges_hbm, out_hbm, acc):
    cid = lax.axis_index("c")
    sid = lax.axis_index("s")
    wid = sid * NC + cid

    def _agg(gball, iball, gsem, ssem, isem):
        my_cpw = jnp.where(cid == 0, CPW0, CPW1)
        rows_v = gball.at[pl.ds(0, CHUNK)]

        zeros16 = jnp.zeros((L,), jnp.float32)

        def _zero_row(i, carry):
            for v in range(D // L):
                rows_v[i, pl.ds(v * L, L)] = zeros16
            return carry

        lax.fori_loop(0, CHUNK, _zero_row, 0)
        base = sid * TROWS
        nfull = TROWS // CHUNK
        rem = TROWS - nfull * CHUNK

        def _zero_chunk(i, carry):
            pltpu.sync_copy(rows_v, acc.at[pl.ds(base + i * CHUNK, CHUNK)])
            return carry

        lax.fori_loop(0, nfull, _zero_chunk, 0)
        if rem:
            pltpu.sync_copy(rows_v.at[pl.ds(0, rem)],
                            acc.at[pl.ds(base + nfull * CHUNK, rem)])

        @pl.when(sid == 0)
        def _zero_tail():
            pltpu.sync_copy(rows_v.at[pl.ds(0, TAIL)],
                            acc.at[pl.ds(NS * TROWS, TAIL)])

        plsc.subcore_barrier()

        pltpu.async_copy(edges_hbm.at[wid, 0], iball.at[pl.ds(0, 3)],
                         isem.at[0])
        pltpu.async_copy(edges_hbm.at[wid, 1], iball.at[pl.ds(3, 3)],
                         isem.at[1])

        PD = NBUF - 1

        def _iter(jo, carry):
            slot = lax.rem(jo, NBUF)

            @pl.when(jo >= NBUF)
            def _wait_scatter():
                ks = jo - NBUF
                iks = lax.rem(ks, NIB)
                pltpu.make_async_copy(
                    gball.at[pl.ds(slot * CHUNK, CHUNK)],
                    acc.at[iball.at[iks * 3 + ROW]], ssem.at[slot]).wait()

            @pl.when(jo + 2 < my_cpw)
            def _prefetch_idx():
                inx = lax.rem(jo + 2, NIB)
                pltpu.async_copy(edges_hbm.at[wid, jo + 2],
                                 iball.at[pl.ds(inx * 3, 3)], isem.at[inx])

            @pl.when(jo < my_cpw)
            def _issue_gather():
                islot = lax.rem(jo, NIB)
                pltpu.make_async_copy(
                    edges_hbm.at[wid, jo], iball.at[pl.ds(islot * 3, 3)],
                    isem.at[islot]).wait()
                pltpu.async_copy(x_hbm.at[iball.at[islot * 3 + COL]],
                                 gball.at[pl.ds(slot * CHUNK, CHUNK)],
                                 gsem.at[slot])

            @pl.when(jnp.logical_and(jo >= PD, jo < my_cpw + PD))
            def _process():
                k = jo - PD
                slotp = lax.rem(k, NBUF)
                ip = lax.rem(k, NIB)
                pbase = slotp * CHUNK
                pltpu.make_async_copy(
                    x_hbm.at[iball.at[ip * 3 + COL]],
                    gball.at[pl.ds(pbase, CHUNK)], gsem.at[slotp]).wait()

                def _scale16(g, c2):
                    evg = lax.bitcast_convert_type(
                        iball[ip * 3 + EV, pl.ds(g * L, L)], jnp.float32)
                    for e in range(L):
                        bv = jnp.take_along_axis(
                            evg, jnp.full((L,), e, jnp.int32),
                            axis=0, mode="promise_in_bounds")
                        r = pbase + g * L + e
                        for v in range(D // L):
                            sl = pl.ds(v * L, L)
                            gball[r, sl] = gball[r, sl] * bv
                    return c2

                lax.fori_loop(0, CHUNK // L, _scale16, 0)

                pltpu.async_copy(gball.at[pl.ds(pbase, CHUNK)],
                                 acc.at[iball.at[ip * 3 + ROW]],
                                 ssem.at[slotp], add=True)

            return carry

        lax.fori_loop(0, my_cpw + NBUF, _iter, 0)
        plsc.subcore_barrier()

        obuf = gball.at[pl.ds(0, CHUNK)]

        def _out_chunk(i, carry):
            off = base + i * CHUNK
            pltpu.sync_copy(acc.at[pl.ds(off, CHUNK)], obuf)
            pltpu.sync_copy(obuf, out_hbm.at[cid, pl.ds(off, CHUNK)])
            return carry

        lax.fori_loop(0, nfull, _out_chunk, 0)
        if rem:
            off = base + nfull * CHUNK
            pltpu.sync_copy(acc.at[pl.ds(off, rem)], obuf.at[pl.ds(0, rem)])
            pltpu.sync_copy(obuf.at[pl.ds(0, rem)],
                            out_hbm.at[cid, pl.ds(off, rem)])

        @pl.when(sid == 0)
        def _out_tail():
            pltpu.sync_copy(acc.at[pl.ds(NS * TROWS, TAIL)],
                            obuf.at[pl.ds(0, TAIL)])
            pltpu.sync_copy(obuf.at[pl.ds(0, TAIL)],
                            out_hbm.at[cid, pl.ds(NS * TROWS, TAIL)])

    pl.run_scoped(
        _agg,
        pltpu.VMEM((NBUF * CHUNK, D), jnp.float32),
        pltpu.VMEM((NIB * 3, CHUNK), jnp.int32),
        pltpu.SemaphoreType.DMA((NBUF,)),
        pltpu.SemaphoreType.DMA((NBUF,)),
        pltpu.SemaphoreType.DMA((NIB,)),
    )


@functools.cache
def _sc_agg():
    return pl.kernel(
        _sc_body,
        out_type=jax.ShapeDtypeStruct((NC, N_NODES, D), jnp.float32),
        mesh=plsc.VectorSubcoreMesh(core_axis_name="c", subcore_axis_name="s",
                                    num_cores=NC, num_subcores=NS),
        scratch_types=[
            pltpu.VMEM_SHARED((N_NODES, D), jnp.float32),
        ],
    )


def _combine_body(p_ref, w_ref, o_ref):
    s = p_ref[0] + p_ref[1]
    o_ref[...] = jnp.maximum(
        jnp.dot(s, w_ref[...], preferred_element_type=jnp.float32), 0.0)


BM = 1000

_combine = pl.pallas_call(
    _combine_body,
    grid=(N_NODES // BM,),
    in_specs=[
        pl.BlockSpec((NC, BM, D), lambda i: (0, i, 0)),
        pl.BlockSpec((D, D), lambda i: (0, 0)),
    ],
    out_specs=pl.BlockSpec((BM, D), lambda i: (i, 0)),
    out_shape=jax.ShapeDtypeStruct((N_NODES, D), jnp.float32),
)


@jax.jit
def kernel(inputs, W, edge_index, edge_vals):
    row = edge_index[0]
    col = edge_index[1]
    pad = E_PAD - N_EDGES
    row_p = jnp.concatenate([row, jnp.zeros((pad,), jnp.int32)])
    col_p = jnp.concatenate([col, jnp.zeros((pad,), jnp.int32)])
    ev_p = jnp.concatenate([edge_vals, jnp.zeros((pad,), jnp.float32)])
    ev_i = lax.bitcast_convert_type(ev_p, jnp.int32)

    def _segment(arr):
        parts, off = [], 0
        for w in range(NW):
            seg = lax.dynamic_slice_in_dim(arr, off, _SEGS[w])
            parts.append(jnp.pad(seg, (0, CPW_MAX * CHUNK - _SEGS[w])))
            off += _SEGS[w]
        return jnp.stack(parts).reshape(NW, CPW_MAX, CHUNK)

    edges = jnp.stack(
        [_segment(col_p), _segment(row_p), _segment(ev_i)], axis=2)
    partials = _sc_agg()(inputs, edges)
    return _combine(partials, W[0])


# speedup vs baseline: 1.1061x; 1.1061x over previous
import functools

import jax
import jax.numpy as jnp
from jax import lax
from jax.experimental import pallas as pl
from jax.experimental.pallas import tpu as pltpu
from jax.experimental.pallas import tpu_sc as plsc

N_NODES = 10000
D = 128
N_EDGES = 320000

NC = 2
NS = 16
L = 16
NW = NC * NS

CHUNK = 128
NBUF = 3
NIB = 5
COL, ROW, EV = 0, 1, 2
CPW0 = 88
CPW1 = 70
CPW_MAX = max(CPW0, CPW1)
_SEGS = [CPW0 * CHUNK if w % NC == 0 else CPW1 * CHUNK for w in range(NW)]
E_PAD = sum(_SEGS)
assert E_PAD >= N_EDGES

TROWS = (N_NODES // NS) // 8 * 8
TAIL = N_NODES - NS * TROWS


def _sc_body(x_hbm, edges_hbm, out_hbm, acc):
    cid = lax.axis_index("c")
    sid = lax.axis_index("s")
    wid = sid * NC + cid

    def _agg(gball, iball, gsem, ssem, isem):
        my_cpw = jnp.where(cid == 0, CPW0, CPW1)
        rows_v = gball.at[pl.ds(0, CHUNK)]

        zeros16 = jnp.zeros((L,), jnp.float32)

        def _zero_row(i, carry):
            for v in range(D // L):
                rows_v[i, pl.ds(v * L, L)] = zeros16
            return carry

        lax.fori_loop(0, CHUNK, _zero_row, 0)
        base = sid * TROWS
        nfull = TROWS // CHUNK
        rem = TROWS - nfull * CHUNK

        def _zero_chunk(i, carry):
            pltpu.sync_copy(rows_v, acc.at[pl.ds(base + i * CHUNK, CHUNK)])
            return carry

        lax.fori_loop(0, nfull, _zero_chunk, 0)
        if rem:
            pltpu.sync_copy(rows_v.at[pl.ds(0, rem)],
                            acc.at[pl.ds(base + nfull * CHUNK, rem)])

        @pl.when(sid == 0)
        def _zero_tail():
            pltpu.sync_copy(rows_v.at[pl.ds(0, TAIL)],
                            acc.at[pl.ds(NS * TROWS, TAIL)])

        plsc.subcore_barrier()

        pltpu.async_copy(edges_hbm.at[wid, 0], iball.at[pl.ds(0, 3)],
                         isem.at[0])
        pltpu.async_copy(edges_hbm.at[wid, 1], iball.at[pl.ds(3, 3)],
                         isem.at[1])

        PD = NBUF - 1

        def _iter(jo, carry):
            slot = lax.rem(jo, NBUF)

            @pl.when(jo >= NBUF)
            def _wait_scatter():
                ks = jo - NBUF
                iks = lax.rem(ks, NIB)
                pltpu.make_async_copy(
                    gball.at[pl.ds(slot * CHUNK, CHUNK)],
                    acc.at[iball.at[iks * 3 + ROW]], ssem.at[slot]).wait()

            @pl.when(jo + 2 < my_cpw)
            def _prefetch_idx():
                inx = lax.rem(jo + 2, NIB)
                pltpu.async_copy(edges_hbm.at[wid, jo + 2],
                                 iball.at[pl.ds(inx * 3, 3)], isem.at[inx])

            @pl.when(jo < my_cpw)
            def _issue_gather():
                islot = lax.rem(jo, NIB)
                pltpu.make_async_copy(
                    edges_hbm.at[wid, jo], iball.at[pl.ds(islot * 3, 3)],
                    isem.at[islot]).wait()
                pltpu.async_copy(x_hbm.at[iball.at[islot * 3 + COL]],
                                 gball.at[pl.ds(slot * CHUNK, CHUNK)],
                                 gsem.at[slot])

            @pl.when(jnp.logical_and(jo >= PD, jo < my_cpw + PD))
            def _process():
                k = jo - PD
                slotp = lax.rem(k, NBUF)
                ip = lax.rem(k, NIB)
                pbase = slotp * CHUNK
                pltpu.make_async_copy(
                    x_hbm.at[iball.at[ip * 3 + COL]],
                    gball.at[pl.ds(pbase, CHUNK)], gsem.at[slotp]).wait()

                def _scale16(g, c2):
                    evg = lax.bitcast_convert_type(
                        iball[ip * 3 + EV, pl.ds(g * L, L)], jnp.float32)
                    for e in range(L):
                        bv = jnp.take_along_axis(
                            evg, jnp.full((L,), e, jnp.int32),
                            axis=0, mode="promise_in_bounds")
                        r = pbase + g * L + e
                        for v in range(D // L):
                            sl = pl.ds(v * L, L)
                            gball[r, sl] = gball[r, sl] * bv
                    return c2

                lax.fori_loop(0, CHUNK // L, _scale16, 0)

                pltpu.async_copy(gball.at[pl.ds(pbase, CHUNK)],
                                 acc.at[iball.at[ip * 3 + ROW]],
                                 ssem.at[slotp], add=True)

            return carry

        lax.fori_loop(0, my_cpw + NBUF, _iter, 0)
        plsc.subcore_barrier()

        obuf = gball.at[pl.ds(0, CHUNK)]

        def _out_chunk(i, carry):
            off = base + i * CHUNK
            pltpu.sync_copy(acc.at[pl.ds(off, CHUNK)], obuf)
            pltpu.sync_copy(obuf, out_hbm.at[cid, pl.ds(off, CHUNK)])
            return carry

        lax.fori_loop(0, nfull, _out_chunk, 0)
        if rem:
            off = base + nfull * CHUNK
            pltpu.sync_copy(acc.at[pl.ds(off, rem)], obuf.at[pl.ds(0, rem)])
            pltpu.sync_copy(obuf.at[pl.ds(0, rem)],
                            out_hbm.at[cid, pl.ds(off, rem)])

        @pl.when(sid == 0)
        def _out_tail():
            pltpu.sync_copy(acc.at[pl.ds(NS * TROWS, TAIL)],
                            obuf.at[pl.ds(0, TAIL)])
            pltpu.sync_copy(obuf.at[pl.ds(0, TAIL)],
                            out_hbm.at[cid, pl.ds(NS * TROWS, TAIL)])

    pl.run_scoped(
        _agg,
        pltpu.VMEM((NBUF * CHUNK, D), jnp.float32),
        pltpu.VMEM((NIB * 3, CHUNK), jnp.int32),
        pltpu.SemaphoreType.DMA((NBUF,)),
        pltpu.SemaphoreType.DMA((NBUF,)),
        pltpu.SemaphoreType.DMA((NIB,)),
    )


@functools.cache
def _sc_agg():
    return pl.kernel(
        _sc_body,
        out_type=jax.ShapeDtypeStruct((NC, N_NODES, D), jnp.float32),
        mesh=plsc.VectorSubcoreMesh(core_axis_name="c", subcore_axis_name="s",
                                    num_cores=NC, num_subcores=NS),
        scratch_types=[
            pltpu.VMEM_SHARED((N_NODES, D), jnp.float32),
        ],
    )


def _combine_body(p_ref, w_ref, o_ref):
    s = p_ref[0] + p_ref[1]
    o_ref[...] = jnp.maximum(
        jnp.dot(s, w_ref[...], preferred_element_type=jnp.float32), 0.0)


BM = 1000

_combine = pl.pallas_call(
    _combine_body,
    grid=(N_NODES // BM,),
    in_specs=[
        pl.BlockSpec((NC, BM, D), lambda i: (0, i, 0)),
        pl.BlockSpec((D, D), lambda i: (0, 0)),
    ],
    out_specs=pl.BlockSpec((BM, D), lambda i: (i, 0)),
    out_shape=jax.ShapeDtypeStruct((N_NODES, D), jnp.float32),
)


@jax.jit
def kernel(inputs, W, edge_index, edge_vals):
    row = edge_index[0]
    col = edge_index[1]
    pad = E_PAD - N_EDGES
    row_p = jnp.concatenate([row, jnp.zeros((pad,), jnp.int32)])
    col_p = jnp.concatenate([col, jnp.zeros((pad,), jnp.int32)])
    ev_p = jnp.concatenate([edge_vals, jnp.zeros((pad,), jnp.float32)])
    ev_i = lax.bitcast_convert_type(ev_p, jnp.int32)

    def _segment(arr):
        parts, off = [], 0
        for w in range(NW):
            seg = lax.dynamic_slice_in_dim(arr, off, _SEGS[w])
            parts.append(jnp.pad(seg, (0, CPW_MAX * CHUNK - _SEGS[w])))
            off += _SEGS[w]
        return jnp.stack(parts).reshape(NW, CPW_MAX, CHUNK)

    edges = jnp.stack(
        [_segment(col_p), _segment(row_p), _segment(ev_i)], axis=2)
    partials = _sc_agg()(inputs, edges)
    return _combine(partials, W[0])
